# baseline (speedup 1.0000x reference)
import functools

import jax
import jax.numpy as jnp
from jax import lax
from jax.experimental import pallas as pl
from jax.experimental.pallas import tpu as pltpu
from jax.experimental.pallas import tpu_sc as plsc

NC = 2
NS = 16
NW = NC * NS

CH = 32
NBUF = 3


def _sc_gather(x_flat, table, D):
    Nsc = x_flat.shape[0]
    b_per_w = Nsc // NW
    n_ch = b_per_w // CH

    mesh = plsc.VectorSubcoreMesh(core_axis_name="c", subcore_axis_name="s")

    @functools.partial(
        pl.kernel,
        mesh=mesh,
        out_type=jax.ShapeDtypeStruct((Nsc, D), jnp.float32),
        scratch_types=[
            pltpu.VMEM((n_ch, CH), jnp.int32),
            pltpu.VMEM((NBUF, CH, D), jnp.float32),
            pltpu.SemaphoreType.DMA,
            pltpu.SemaphoreType.DMA,
        ],
    )
    def gather_k(idx_hbm, table_hbm, out_hbm, idx_v, rows_v, gsem, ssem):
        wid = lax.axis_index("s") * NC + lax.axis_index("c")
        base = wid * b_per_w
        pltpu.sync_copy(idx_hbm.at[wid], idx_v)

        def start_gather(j, b):
            pltpu.async_copy(table_hbm.at[idx_v.at[j]], rows_v.at[b], gsem)

        def wait_gather(b):
            pltpu.make_async_copy(
                table_hbm.at[idx_v.at[0]], rows_v.at[b], gsem
            ).wait()

        def start_scatter(j, b):
            pltpu.async_copy(
                rows_v.at[b], out_hbm.at[pl.ds(base + j * CH, CH)], ssem
            )

        def wait_scatter(b):
            pltpu.make_async_copy(
                rows_v.at[b], out_hbm.at[pl.ds(base, CH)], ssem
            ).wait()

        for j in range(NBUF - 1):
            start_gather(j, j)
        wait_gather(0)
        start_scatter(0, 0)
        start_gather(NBUF - 1, NBUF - 1)

        def group(g, carry):
            j0 = NBUF * g + 1
            for k in range(NBUF):
                b = (1 + k) % NBUF
                j = j0 + k
                wait_gather(b)
                start_scatter(j, b)
                wait_scatter((b + NBUF - 1) % NBUF)
                start_gather(j + NBUF - 1, (b + NBUF - 1) % NBUF)
            return carry

        n_steady = n_ch - NBUF
        lax.fori_loop(0, n_steady // NBUF, group, 0)

        for j in range(1 + NBUF * (n_steady // NBUF), n_ch - NBUF + 1):
            b = j % NBUF
            wait_gather(b)
            start_scatter(j, b)
            wait_scatter((b + NBUF - 1) % NBUF)
            start_gather(j + NBUF - 1, (b + NBUF - 1) % NBUF)

        for j in range(n_ch - NBUF + 1, n_ch):
            b = j % NBUF
            wait_gather(b)
            start_scatter(j, b)
            wait_scatter((b + NBUF - 1) % NBUF)
        wait_scatter((n_ch - 1) % NBUF)

    idx = x_flat.reshape(NW, n_ch, CH).astype(jnp.int32)
    return gather_k(idx, table)


@jax.jit
def kernel(x, table):
    B, S = x.shape
    V, D = table.shape
    N = B * S
    split = (N * 3) // 4
    x_flat = x.reshape(N)
    sc_out = _sc_gather(x_flat[:split], table, D)
    tc_out = jnp.take(table, x_flat[split:], axis=0)
    return jnp.concatenate([sc_out, tc_out], axis=0).reshape(B, S, D)


# speedup vs baseline: 1.9772x; 1.9772x over previous
import functools

import jax
import jax.numpy as jnp
from jax import lax
from jax.experimental import pallas as pl
from jax.experimental.pallas import tpu as pltpu
from jax.experimental.pallas import tpu_sc as plsc

NC = 2
NS = 16
NW = NC * NS

CH = 32
NBUF = 3


@jax.jit
def kernel(x, table):
    B, S = x.shape
    V, D = table.shape
    N = B * S
    assert N % NW == 0
    b_per_w = N // NW
    assert b_per_w % CH == 0
    n_ch = b_per_w // CH
    assert n_ch >= 2 * NBUF

    mesh = plsc.VectorSubcoreMesh(core_axis_name="c", subcore_axis_name="s")

    @functools.partial(
        pl.kernel,
        mesh=mesh,
        out_type=jax.ShapeDtypeStruct((N, D), jnp.float32),
        scratch_types=[
            pltpu.VMEM((n_ch, CH), jnp.int32),
            pltpu.VMEM((NBUF, CH, D), jnp.float32),
            pltpu.SemaphoreType.DMA,
            pltpu.SemaphoreType.DMA,
        ],
    )
    def gather_k(idx_hbm, table_hbm, out_hbm, idx_v, rows_v, gsem, ssem):
        wid = lax.axis_index("s") * NC + lax.axis_index("c")
        base = wid * b_per_w
        pltpu.sync_copy(idx_hbm.at[wid], idx_v)

        def start_gather(j, b):
            pltpu.async_copy(table_hbm.at[idx_v.at[j]], rows_v.at[b], gsem)

        def wait_gather(b):
            pltpu.make_async_copy(
                table_hbm.at[idx_v.at[0]], rows_v.at[b], gsem
            ).wait()

        def start_scatter(j, b):
            pltpu.async_copy(
                rows_v.at[b], out_hbm.at[pl.ds(base + j * CH, CH)], ssem
            )

        def wait_scatter(b):
            pltpu.make_async_copy(
                rows_v.at[b], out_hbm.at[pl.ds(base, CH)], ssem
            ).wait()

        for j in range(NBUF - 1):
            start_gather(j, j)
        wait_gather(0)
        start_scatter(0, 0)
        start_gather(NBUF - 1, NBUF - 1)

        def group(g, carry):
            j0 = NBUF * g + 1
            for k in range(NBUF):
                b = (1 + k) % NBUF
                j = j0 + k
                wait_gather(b)
                start_scatter(j, b)
                wait_scatter((b + NBUF - 1) % NBUF)
                start_gather(j + NBUF - 1, (b + NBUF - 1) % NBUF)
            return carry

        n_steady = n_ch - NBUF
        lax.fori_loop(0, n_steady // NBUF, group, 0)

        for j in range(1 + NBUF * (n_steady // NBUF), n_ch - NBUF + 1):
            b = j % NBUF
            wait_gather(b)
            start_scatter(j, b)
            wait_scatter((b + NBUF - 1) % NBUF)
            start_gather(j + NBUF - 1, (b + NBUF - 1) % NBUF)

        for j in range(n_ch - NBUF + 1, n_ch):
            b = j % NBUF
            wait_gather(b)
            start_scatter(j, b)
            wait_scatter((b + NBUF - 1) % NBUF)
        wait_scatter((n_ch - 1) % NBUF)

    idx = x.reshape(NW, n_ch, CH).astype(jnp.int32)
    out = gather_k(idx, table)
    return out.reshape(B, S, D)
